# Initial kernel scaffold; baseline (speedup 1.0000x reference)
import jax
import jax.numpy as jnp
from jax.experimental import pallas as pl


def kernel(edge_attr, W0, W1, W2):
    raise NotImplementedError("write your pallas kernel here")


# speedup vs baseline: 2.6150x; 2.6150x over previous
import functools

import jax
import jax.numpy as jnp
from jax import lax
from jax.experimental import pallas as pl
from jax.experimental.pallas import tpu as pltpu
from jax.experimental.pallas import tpu_sc as plsc

_NC = 2
_NS = 16
_NW = _NC * _NS
_LANES = 16


def _combine_tables_body(w0_ref, w1_ref, w2_ref, t_ref):
    rows = t_ref.shape[0]

    def onehot(n, sel):
        r = lax.broadcasted_iota(jnp.int32, (rows, n), 0)
        c = lax.broadcasted_iota(jnp.int32, (rows, n), 1)
        return (sel(r) == c).astype(jnp.float32)

    hi = jax.lax.Precision.HIGHEST
    t = jnp.dot(onehot(w0_ref.shape[0], lambda r: r // 32), w0_ref[:],
                preferred_element_type=jnp.float32, precision=hi)
    t += jnp.dot(onehot(w1_ref.shape[0], lambda r: (r // 4) % 8), w1_ref[:],
                 preferred_element_type=jnp.float32, precision=hi)
    t += jnp.dot(onehot(w2_ref.shape[0], lambda r: r % 4), w2_ref[:],
                 preferred_element_type=jnp.float32, precision=hi)
    t_ref[:] = t


def _build_table(W0, W1, W2):
    rows = 7 * 8 * 4
    return pl.pallas_call(
        _combine_tables_body,
        out_shape=jax.ShapeDtypeStruct((rows, W0.shape[1]), jnp.float32),
    )(W0, W1, W2)


def _make_lookup(E, H):
    PW = E // _NW
    assert E % _NW == 0
    CH = 128
    NCH = PW // CH
    TAIL = PW - NCH * CH
    NV = (PW + _LANES - 1) // _LANES
    EA_PAD = NV * _LANES * 3

    mesh = plsc.VectorSubcoreMesh(core_axis_name="c", subcore_axis_name="s")

    @functools.partial(
        pl.kernel,
        out_type=jax.ShapeDtypeStruct((E, H), jnp.float32),
        mesh=mesh,
        compiler_params=pltpu.CompilerParams(needs_layout_passes=False),
        scratch_types=[
            pltpu.VMEM((EA_PAD,), jnp.int32),
            pltpu.VMEM((NV * _LANES,), jnp.int32),
            pltpu.VMEM((CH, H), jnp.float32),
            pltpu.SemaphoreType.DMA,
        ],
    )
    def lookup(t_hbm, ea_hbm, out_hbm, ea_v, c_v, buf, sem):
        wid = lax.axis_index("s") * _NC + lax.axis_index("c")
        base = wid * PW

        pltpu.sync_copy(ea_hbm.at[pl.ds(base * 3, PW * 3)],
                        ea_v.at[pl.ds(0, PW * 3)])

        def idx_step(j, carry):
            lanes = lax.iota(jnp.int32, _LANES) * 3 + j * (3 * _LANES)
            a0 = plsc.load_gather(ea_v, [lanes])
            a1 = plsc.load_gather(ea_v, [lanes + 1])
            a2 = plsc.load_gather(ea_v, [lanes + 2])
            a0 = jnp.clip(a0, 0, 6)
            a1 = jnp.clip(a1, 0, 7)
            a2 = jnp.clip(a2, 0, 3)
            c_v[pl.ds(j * _LANES, _LANES)] = a0 * 32 + a1 * 4 + a2
            return carry

        lax.fori_loop(0, NV, idx_step, 0)

        def chunk_step(i, carry):
            pltpu.async_copy(t_hbm.at[c_v.at[pl.ds(i * CH, CH)]],
                             buf, sem).wait()
            pltpu.sync_copy(buf, out_hbm.at[pl.ds(base + i * CH, CH)])
            return carry

        lax.fori_loop(0, NCH, chunk_step, 0)

        if TAIL:
            pltpu.async_copy(t_hbm.at[c_v.at[pl.ds(NCH * CH, TAIL)]],
                             buf.at[pl.ds(0, TAIL)], sem).wait()
            pltpu.sync_copy(buf.at[pl.ds(0, TAIL)],
                            out_hbm.at[pl.ds(base + NCH * CH, TAIL)])

    return lookup


def kernel(edge_attr, W0, W1, W2):
    E = edge_attr.shape[0]
    H = W0.shape[1]
    table = _build_table(W0, W1, W2)
    ea_flat = edge_attr.astype(jnp.int32).reshape(-1)
    return _make_lookup(E, H)(table, ea_flat)
